# baseline (speedup 1.0000x reference)
import jax
import jax.numpy as jnp
from jax.experimental import pallas as pl
from jax.experimental.pallas import tpu as pltpu

_VMEM = pl.BlockSpec(memory_space=pltpu.MemorySpace.VMEM)
_ANY = pl.BlockSpec(memory_space=pl.ANY)


def _make_fused(B, Cout, Ho, Wo, Hd, WC, G):
    P = Ho * Wo
    L = 3 * G

    def _fused(x_ref, m_ref, bias_ref, vw_ref, vb_ref,
               glove_ref, tw_ref, tb_ref, h0_ref, c0_ref,
               wihv_ref, wiht_ref, whh_ref, bl_ref, acw_ref, acb_ref,
               h_out_ref, c_out_ref, pv_ref,
               pad_ref, vw_scr, wv_scr, wt_scr, wh_scr, sems):
        cps = []
        for c in range(Cout):
            cp = pltpu.make_async_copy(vw_ref.at[c], vw_scr.at[c], sems.at[c])
            cp.start()
            cps.append(cp)
        cp_v = pltpu.make_async_copy(wihv_ref, wv_scr, sems.at[Cout])
        cp_t = pltpu.make_async_copy(wiht_ref, wt_scr, sems.at[Cout + 1])
        cp_h = pltpu.make_async_copy(whh_ref, wh_scr, sems.at[Cout + 2])
        cp_v.start()
        cp_t.start()
        cp_h.start()

        bf = jnp.bfloat16
        pad_ref[:, 0:1, 0:L] = jnp.zeros((B, 1, L), bf)
        pad_ref[:, 1:Ho + 1, 0:3] = jnp.zeros((B, Ho, 3), bf)
        pad_ref[:, 0:Ho, L:L + 3] = jnp.zeros((B, Ho, 3), bf)
        pad_ref[:, 0:Ho, L + 3:L + 3 + WC] = x_ref[:, :, 0:WC].astype(bf)
        pad_ref[:, 1:Ho + 1, 3:3 + WC] = x_ref[:, :, WC:2 * WC].astype(bf)

        a0 = pad_ref[:, 0:Ho, 0:L].reshape(B * Ho, L)
        a1 = pad_ref[:, 0:Ho, L:2 * L].reshape(B * Ho, L)
        a2 = pad_ref[:, 1:Ho + 1, 0:L].reshape(B * Ho, L)
        cp = (jnp.dot(a0, m_ref[0], preferred_element_type=jnp.float32)
              + jnp.dot(a1, m_ref[1], preferred_element_type=jnp.float32)
              + jnp.dot(a2, m_ref[2], preferred_element_type=jnp.float32))
        conv = jnp.maximum(cp.reshape(B, Ho, Cout * Wo) + bias_ref[...], 0.0)

        acc = jnp.zeros((B, Hd), jnp.float32)
        for c in range(Cout):
            cc = conv[:, :, c * Wo:(c + 1) * Wo].reshape(B, P)
            cps[c].wait()
            acc = acc + jnp.dot(cc, vw_scr[c],
                                preferred_element_type=jnp.float32)
        ve = jnp.maximum(acc + vb_ref[...], 0.0)
        te = jnp.maximum(jnp.dot(glove_ref[...], tw_ref[...],
                                 preferred_element_type=jnp.float32)
                         + tb_ref[...], 0.0)
        cp_v.wait()
        cp_t.wait()
        cp_h.wait()
        gates = (jnp.dot(ve, wv_scr[...],
                         preferred_element_type=jnp.float32)
                 + jnp.dot(te, wt_scr[...],
                           preferred_element_type=jnp.float32)
                 + jnp.dot(h0_ref[...], wh_scr[...],
                           preferred_element_type=jnp.float32)
                 + bl_ref[...])
        i = jax.nn.sigmoid(gates[:, 0 * Hd:1 * Hd])
        f = jax.nn.sigmoid(gates[:, 1 * Hd:2 * Hd])
        g = jnp.tanh(gates[:, 2 * Hd:3 * Hd])
        o = jax.nn.sigmoid(gates[:, 3 * Hd:4 * Hd])
        c_new = f * c0_ref[...] + i * g
        h_new = o * jnp.tanh(c_new)
        c_out_ref[...] = c_new
        h_out_ref[...] = h_new
        pv_ref[...] = (jnp.dot(h_new, acw_ref[...],
                               preferred_element_type=jnp.float32)
                       + acb_ref[...])
    return _fused


def kernel(img_pad_value, conv_w_t, conv_b_col, vobs_w, vobs_b, tobs_w, tobs_b,
           lstm_w_ih_v, lstm_w_ih_t, lstm_w_hh, lstm_b, ac_w, ac_b,
           image, glove, h0, c0):
    B, H, W, C = image.shape
    Ho, Wo = H // 2, W // 2
    P = Ho * Wo
    Cout, K = conv_w_t.shape
    Hd = h0.shape[1]
    Dg = lstm_w_hh.shape[1]
    A1 = ac_w.shape[1]
    bf = jnp.bfloat16
    WC = W * C
    G = W + 1
    L = 3 * G

    ximg = image.reshape(B, Ho, 2 * WC)

    jdx = jnp.arange(Wo)
    M = jnp.zeros((3, L, Cout * Wo), jnp.float32)
    for di in range(3):
        acc = jnp.zeros((G, 3, Cout, Wo), jnp.float32)
        for dj in range(3):
            ind = jax.nn.one_hot(2 * jdx + dj, G, axis=0)
            wsl = conv_w_t[:, (di * 3 + dj) * 3:(di * 3 + dj) * 3 + 3]
            acc = acc + jnp.einsum('gj,cd->gdcj', ind, wsl)
        M = M.at[di].set(acc.reshape(L, Cout * Wo))
    M = M.astype(bf)

    ii = jnp.arange(Ho)
    oob_rows = [((2 * ii + di - 1 < 0) | (2 * ii + di - 1 >= H))
                for di in range(3)]
    oob_cols = [((2 * jdx + dj - 1 < 0) | (2 * jdx + dj - 1 >= W))
                for dj in range(3)]
    oob = jnp.stack([oob_rows[di][:, None] | oob_cols[dj][None, :]
                     for di in range(3) for dj in range(3)])
    oob27 = jnp.repeat(oob.reshape(9, P).astype(jnp.float32), C, axis=0)
    padv = jnp.tile(img_pad_value.reshape(-1), 9)
    bias = conv_b_col + (conv_w_t * padv[None, :]) @ oob27
    biasN = bias.reshape(Cout, Ho, Wo).transpose(1, 0, 2).reshape(1, Ho, Cout * Wo)

    vw3 = vobs_w.reshape(Cout, P, Hd)

    h_new, c_new, pv = pl.pallas_call(
        _make_fused(B, Cout, Ho, Wo, Hd, WC, G),
        out_shape=(jax.ShapeDtypeStruct((B, Hd), jnp.float32),
                   jax.ShapeDtypeStruct((B, Hd), jnp.float32),
                   jax.ShapeDtypeStruct((B, A1), jnp.float32)),
        in_specs=[_VMEM, _VMEM, _VMEM, _ANY, _VMEM,
                  _VMEM, _VMEM, _VMEM, _VMEM, _VMEM,
                  _ANY, _ANY, _ANY, _VMEM, _VMEM, _VMEM],
        out_specs=(_VMEM, _VMEM, _VMEM),
        scratch_shapes=[pltpu.VMEM((B, Ho + 1, 2 * L), bf),
                        pltpu.VMEM((Cout, P, Hd), jnp.float32),
                        pltpu.VMEM((Hd, Dg), jnp.float32),
                        pltpu.VMEM((Hd, Dg), jnp.float32),
                        pltpu.VMEM((Hd, Dg), jnp.float32),
                        pltpu.SemaphoreType.DMA((Cout + 3,))],
    )(ximg, M, biasN, vw3, vobs_b,
      glove, tobs_w, tobs_b, h0, c0,
      lstm_w_ih_v, lstm_w_ih_t, lstm_w_hh,
      lstm_b, ac_w, ac_b)

    A = A1 - 1
    return {'policy': pv[:, :A], 'value': pv[:, A:], 'hidden': (h_new, c_new)}


# speedup vs baseline: 1.1212x; 1.1212x over previous
import jax
import jax.numpy as jnp
from jax.experimental import pallas as pl
from jax.experimental.pallas import tpu as pltpu

_VMEM = pl.BlockSpec(memory_space=pltpu.MemorySpace.VMEM)
_ANY = pl.BlockSpec(memory_space=pl.ANY)


def _make_fused(B, Cout, Ho, Wo, Hd, WC, G, T_in, A1):
    P = Ho * Wo
    L = 3 * G
    r_tw = B
    r_h0 = r_tw + T_in
    r_c0 = r_h0 + B
    r_vb = r_c0 + B
    r_tb = r_vb + 1
    r_bl = r_tb + 1
    r_acb = r_bl + 4
    r_acw = r_acb + 1
    r_bias = r_acw + Hd

    def _fused(x_ref, m_ref, pk_ref, vw_ref,
               wihv_ref, wiht_ref, whh_ref,
               out_ref, pad_ref):
        bf = jnp.bfloat16
        pad_ref[:, 0:1, 0:L] = jnp.zeros((B, 1, L), bf)
        pad_ref[:, 1:Ho + 1, 0:3] = jnp.zeros((B, Ho, 3), bf)
        pad_ref[:, 0:Ho, L:L + 3] = jnp.zeros((B, Ho, 3), bf)
        pad_ref[:, 0:Ho, L + 3:L + 3 + WC] = x_ref[:, :, 0:WC]
        pad_ref[:, 1:Ho + 1, 3:3 + WC] = x_ref[:, :, WC:2 * WC]

        a0 = pad_ref[:, 0:Ho, 0:L].reshape(B * Ho, L)
        a1 = pad_ref[:, 0:Ho, L:2 * L].reshape(B * Ho, L)
        a2 = pad_ref[:, 1:Ho + 1, 0:L].reshape(B * Ho, L)
        cp = (jnp.dot(a0, m_ref[0], preferred_element_type=jnp.float32)
              + jnp.dot(a1, m_ref[1], preferred_element_type=jnp.float32)
              + jnp.dot(a2, m_ref[2], preferred_element_type=jnp.float32))
        conv = jnp.maximum(cp.reshape(B, Ho, Cout * Wo)
                           + pk_ref[r_bias:r_bias + Ho, 0:Cout * Wo][None], 0.0)

        acc = jnp.zeros((B, Hd), jnp.float32)
        for c in range(Cout):
            cc = conv[:, :, c * Wo:(c + 1) * Wo].reshape(B, P)
            acc = acc + jnp.dot(cc, vw_ref[c],
                                preferred_element_type=jnp.float32)
        ve = jnp.maximum(acc + pk_ref[r_vb:r_vb + 1, :], 0.0)
        te = jnp.maximum(jnp.dot(pk_ref[0:B, 0:T_in],
                                 pk_ref[r_tw:r_tw + T_in, :],
                                 preferred_element_type=jnp.float32)
                         + pk_ref[r_tb:r_tb + 1, :], 0.0)
        bl = pk_ref[r_bl:r_bl + 4, :].reshape(1, 4 * Hd)
        gates = (jnp.dot(ve, wihv_ref[...],
                         preferred_element_type=jnp.float32)
                 + jnp.dot(te, wiht_ref[...],
                           preferred_element_type=jnp.float32)
                 + jnp.dot(pk_ref[r_h0:r_h0 + B, :], whh_ref[...],
                           preferred_element_type=jnp.float32)
                 + bl)
        i = jax.nn.sigmoid(gates[:, 0 * Hd:1 * Hd])
        f = jax.nn.sigmoid(gates[:, 1 * Hd:2 * Hd])
        g = jnp.tanh(gates[:, 2 * Hd:3 * Hd])
        o = jax.nn.sigmoid(gates[:, 3 * Hd:4 * Hd])
        c_new = f * pk_ref[r_c0:r_c0 + B, :] + i * g
        h_new = o * jnp.tanh(c_new)
        out_ref[:, 0:Hd] = h_new
        out_ref[:, Hd:2 * Hd] = c_new
        out_ref[:, 2 * Hd:2 * Hd + A1] = (
            jnp.dot(h_new, pk_ref[r_acw:r_acw + Hd, 0:A1],
                    preferred_element_type=jnp.float32)
            + pk_ref[r_acb:r_acb + 1, 0:A1])
    return _fused


def kernel(img_pad_value, conv_w_t, conv_b_col, vobs_w, vobs_b, tobs_w, tobs_b,
           lstm_w_ih_v, lstm_w_ih_t, lstm_w_hh, lstm_b, ac_w, ac_b,
           image, glove, h0, c0):
    B, H, W, C = image.shape
    Ho, Wo = H // 2, W // 2
    P = Ho * Wo
    Cout, K = conv_w_t.shape
    Hd = h0.shape[1]
    Dg = lstm_w_hh.shape[1]
    T_in = glove.shape[1]
    A1 = ac_w.shape[1]
    bf = jnp.bfloat16
    WC = W * C
    G = W + 1
    L = 3 * G

    ximg = image.reshape(B, Ho, 2 * WC).astype(bf)

    jdx = jnp.arange(Wo)
    M = jnp.zeros((3, L, Cout * Wo), jnp.float32)
    for di in range(3):
        acc = jnp.zeros((G, 3, Cout, Wo), jnp.float32)
        for dj in range(3):
            ind = jax.nn.one_hot(2 * jdx + dj, G, axis=0)
            wsl = conv_w_t[:, (di * 3 + dj) * 3:(di * 3 + dj) * 3 + 3]
            acc = acc + jnp.einsum('gj,cd->gdcj', ind, wsl)
        M = M.at[di].set(acc.reshape(L, Cout * Wo))
    M = M.astype(bf)

    ii = jnp.arange(Ho)
    oob_rows = [((2 * ii + di - 1 < 0) | (2 * ii + di - 1 >= H))
                for di in range(3)]
    oob_cols = [((2 * jdx + dj - 1 < 0) | (2 * jdx + dj - 1 >= W))
                for dj in range(3)]
    oob = jnp.stack([oob_rows[di][:, None] | oob_cols[dj][None, :]
                     for di in range(3) for dj in range(3)])
    oob27 = jnp.repeat(oob.reshape(9, P).astype(jnp.float32), C, axis=0)
    padv = jnp.tile(img_pad_value.reshape(-1), 9)
    bias = conv_b_col + (conv_w_t * padv[None, :]) @ oob27
    biasN = bias.reshape(Cout, Ho, Wo).transpose(1, 0, 2).reshape(Ho, Cout * Wo)

    vw3 = vobs_w.reshape(Cout, P, Hd)

    n_rows = B + T_in + B + B + 1 + 1 + 4 + 1 + Hd + Ho
    pk = jnp.zeros((n_rows, Hd), jnp.float32)
    r = 0
    pk = pk.at[r:r + B, 0:T_in].set(glove); r += B
    pk = pk.at[r:r + T_in, :].set(tobs_w); r += T_in
    pk = pk.at[r:r + B, :].set(h0); r += B
    pk = pk.at[r:r + B, :].set(c0); r += B
    pk = pk.at[r:r + 1, :].set(vobs_b); r += 1
    pk = pk.at[r:r + 1, :].set(tobs_b); r += 1
    pk = pk.at[r:r + 4, :].set(lstm_b.reshape(4, Hd)); r += 4
    pk = pk.at[r:r + 1, 0:A1].set(ac_b); r += 1
    pk = pk.at[r:r + Hd, 0:A1].set(ac_w); r += Hd
    pk = pk.at[r:r + Ho, 0:Cout * Wo].set(biasN); r += Ho

    out = pl.pallas_call(
        _make_fused(B, Cout, Ho, Wo, Hd, WC, G, T_in, A1),
        out_shape=jax.ShapeDtypeStruct((B, 2 * Hd + 128), jnp.float32),
        in_specs=[_VMEM] * 7,
        out_specs=_VMEM,
        scratch_shapes=[pltpu.VMEM((B, Ho + 1, 2 * L), bf)],
    )(ximg, M, pk, vw3, lstm_w_ih_v, lstm_w_ih_t, lstm_w_hh)

    A = A1 - 1
    h_new = out[:, 0:Hd]
    c_new = out[:, Hd:2 * Hd]
    return {'policy': out[:, 2 * Hd:2 * Hd + A],
            'value': out[:, 2 * Hd + A:2 * Hd + A1],
            'hidden': (h_new, c_new)}
